# baseline (speedup 1.0000x reference)
import functools

import jax
import jax.numpy as jnp
from jax import lax
from jax.experimental import pallas as pl
from jax.experimental.pallas import tpu as pltpu
from jax.experimental.pallas import tpu_sc as plsc

N = 16384
C_CAT = 26
C_NUM = 13
D = 16
NC = 2
NS = 16
NW = NC * NS
ROWS_PER_W = N // NW
CHUNK = 128
NCHUNK = ROWS_PER_W // CHUNK


def _sc_body(emb_u, idx_u, num_u, lw_u, emb_i, idx_i, num_i, lw_i,
             out_hbm, idx_v, rows_v, num_v, out_v, lw_v, sem):
    wid = lax.axis_index("s") * NC + lax.axis_index("c")
    base = wid * ROWS_PER_W
    for t in range(2):
        emb, idxt, num, lw = ((emb_u, idx_u, num_u, lw_u) if t == 0
                              else (emb_i, idx_i, num_i, lw_i))
        pltpu.sync_copy(lw, lw_v)
        lwr = [lw_v[c] for c in range(C_NUM + 1)]
        for g in range(NCHUNK):
            row0 = base + g * CHUNK
            pltpu.sync_copy(idxt.at[:, pl.ds(row0, CHUNK)], idx_v)
            handles = [
                pltpu.async_copy(emb.at[idx_v.at[c]], rows_v.at[c], sem)
                for c in range(C_CAT)
            ]
            pltpu.sync_copy(num.at[pl.ds(row0, CHUNK)], num_v)
            for h in handles:
                h.wait()

            def row_body(r, _):
                num_row = num_v[r]
                acc = num_row[0] * lwr[0]
                for c in range(1, C_NUM + 1):
                    acc = acc + num_row[c] * lwr[c]
                for c in range(C_CAT):
                    acc = acc + rows_v[c, r]
                out_v[r] = acc
                return 0

            lax.fori_loop(0, CHUNK, row_body, 0)
            pltpu.sync_copy(out_v, out_hbm.at[t, pl.ds(row0, CHUNK)])


@jax.jit
def _run(emb_u, idx_u, num_u, lw_u, emb_i, idx_i, num_i, lw_i):
    mesh = plsc.VectorSubcoreMesh(core_axis_name="c", subcore_axis_name="s")
    return pl.kernel(
        _sc_body,
        out_type=jax.ShapeDtypeStruct((2, N, D), jnp.float32),
        mesh=mesh,
        scratch_types=[
            pltpu.VMEM((C_CAT, CHUNK), jnp.int32),
            pltpu.VMEM((C_CAT, CHUNK, D), jnp.float32),
            pltpu.VMEM((CHUNK, D), jnp.float32),
            pltpu.VMEM((CHUNK, D), jnp.float32),
            pltpu.VMEM((D, D), jnp.float32),
            pltpu.SemaphoreType.DMA,
        ],
        compiler_params=pltpu.CompilerParams(use_tc_tiling_on_sc=False),
    )(emb_u, idx_u, num_u, lw_u, emb_i, idx_i, num_i, lw_i)


def kernel(cat_user, num_user, cat_item, num_item,
           emb_user, lin_w_user, lin_b_user,
           emb_item, lin_w_item, lin_b_item):
    V = emb_user.shape[1]
    col = (jnp.arange(C_CAT, dtype=jnp.int32) * V)[:, None]
    idx_u = cat_user.astype(jnp.int32).T + col
    idx_i = cat_item.astype(jnp.int32).T + col
    ones = jnp.ones((N, 1), jnp.float32)
    zer = jnp.zeros((N, D - C_NUM - 1), jnp.float32)
    num_u = jnp.concatenate([num_user, ones, zer], axis=1)
    num_i = jnp.concatenate([num_item, ones, zer], axis=1)
    zw = jnp.zeros((D - C_NUM - 1, D), jnp.float32)
    lw_u = jnp.concatenate([lin_w_user, lin_b_user.sum(0)[None], zw], axis=0)
    lw_i = jnp.concatenate([lin_w_item, lin_b_item.sum(0)[None], zw], axis=0)
    return _run(emb_user.reshape(C_CAT * V, D), idx_u, num_u, lw_u,
                emb_item.reshape(C_CAT * V, D), idx_i, num_i, lw_i)


# speedup vs baseline: 4.0224x; 4.0224x over previous
import functools

import jax
import jax.numpy as jnp
from jax import lax
from jax.experimental import pallas as pl
from jax.experimental.pallas import tpu as pltpu
from jax.experimental.pallas import tpu_sc as plsc

N = 16384
C_CAT = 26
C_NUM = 13
V = 100000
D = 16
NC = 2
NS = 16
NW = NC * NS

IDX_P = 8192
NUM_P = 2048


def _sc_body(embT_u, embT_i, idxT, numT, lwb, out2,
             acc_v, idx_v, lw_v, sem, vsem):
    wid = lax.axis_index("s") * NC + lax.axis_index("c")
    t = wid // D
    d = wid % D

    pltpu.sync_copy(lwb.at[wid], lw_v)
    wk = [lw_v[pl.ds(k * D, D)] for k in range(C_NUM + 1)]

    def init_scope(num_v):
        nb = [num_v.at[0], num_v.at[1]]
        cps = [pltpu.async_copy(numT.at[pl.ds(t * D, D), pl.ds(p * NUM_P, NUM_P)],
                                nb[p % 2], vsem)
               for p in range(2)]
        for p in range(N // NUM_P):
            cps[p % 2].wait()
            buf = nb[p % 2]

            def nbody(j, _):
                val = buf[0, pl.ds(j * D, D)] * wk[0]
                for k in range(1, C_NUM + 1):
                    val = val + buf[k, pl.ds(j * D, D)] * wk[k]
                acc_v[pl.ds((p * NUM_P) + j * D, D)] = val
                return 0

            lax.fori_loop(0, NUM_P // D, nbody, 0, unroll=4)
            if p + 2 < N // NUM_P:
                cps[p % 2] = pltpu.async_copy(
                    numT.at[pl.ds(t * D, D), pl.ds((p + 2) * NUM_P, NUM_P)],
                    nb[p % 2], vsem)

    pl.run_scoped(init_scope, pltpu.VMEM((2, D, NUM_P), jnp.float32))

    def main_scope(vec_v):
        for c in range(C_CAT):
            row = c * D + d

            @pl.when(t == 0)
            def _():
                pltpu.sync_copy(embT_u.at[row], vec_v)

            @pl.when(t > 0)
            def _():
                pltpu.sync_copy(embT_i.at[row], vec_v)

            for h in range(N // IDX_P):
                pltpu.sync_copy(idxT.at[t, c, pl.ds(h * IDX_P, IDX_P)], idx_v)
                base = h * IDX_P

                def gbody(j, _):
                    idxv = idx_v[pl.ds(j * D, D)]
                    g = plsc.load_gather(vec_v, [idxv])
                    a = base + j * D
                    acc_v[pl.ds(a, D)] = acc_v[pl.ds(a, D)] + g
                    return 0

                lax.fori_loop(0, IDX_P // D, gbody, 0, unroll=4)

    pl.run_scoped(main_scope, pltpu.VMEM((V,), jnp.float32))

    pltpu.sync_copy(acc_v, out2.at[wid])


@jax.jit
def _run(embT_u, embT_i, idxT, numT, lwb):
    mesh = plsc.VectorSubcoreMesh(core_axis_name="c", subcore_axis_name="s")
    return pl.kernel(
        _sc_body,
        out_type=jax.ShapeDtypeStruct((NW, N), jnp.float32),
        mesh=mesh,
        scratch_types=[
            pltpu.VMEM((N,), jnp.float32),
            pltpu.VMEM((IDX_P,), jnp.int32),
            pltpu.VMEM((D * D,), jnp.float32),
            pltpu.SemaphoreType.DMA,
            pltpu.SemaphoreType.DMA,
        ],
        compiler_params=pltpu.CompilerParams(needs_layout_passes=False),
    )(embT_u, embT_i, idxT, numT, lwb)


def kernel(cat_user, num_user, cat_item, num_item,
           emb_user, lin_w_user, lin_b_user,
           emb_item, lin_w_item, lin_b_item):
    embT_u = emb_user.swapaxes(1, 2).reshape(C_CAT * D, V)
    embT_i = emb_item.swapaxes(1, 2).reshape(C_CAT * D, V)
    idxT = jnp.stack([cat_user.astype(jnp.int32).T,
                      cat_item.astype(jnp.int32).T])
    ones = jnp.ones((1, N), jnp.float32)
    zer = jnp.zeros((D - C_NUM - 1, N), jnp.float32)
    numT = jnp.concatenate(
        [num_user.T, ones, zer, num_item.T, ones, zer], axis=0)
    zw = jnp.zeros((D - C_NUM - 1, D), jnp.float32)
    lw_u = jnp.concatenate([lin_w_user, lin_b_user.sum(0)[None], zw], axis=0)
    lw_i = jnp.concatenate([lin_w_item, lin_b_item.sum(0)[None], zw], axis=0)
    lw2 = jnp.stack([lw_u, lw_i])
    lwb = jnp.repeat(lw2.transpose(0, 2, 1)[:, :, :, None], D, axis=3)
    lwb = lwb.reshape(NW, D * D)
    out2 = _run(embT_u, embT_i, idxT, numT, lwb)
    return out2.reshape(2, D, N).swapaxes(1, 2)
